# Initial kernel scaffold; baseline (speedup 1.0000x reference)
import jax
import jax.numpy as jnp
from jax.experimental import pallas as pl


def kernel(node_feats, edge_feats, edge_index, Wn, We, Wa, Wo):
    raise NotImplementedError("write your pallas kernel here")


# speedup vs baseline: 25.9021x; 25.9021x over previous
import functools
import jax
import jax.numpy as jnp
from jax import lax
from jax.experimental import pallas as pl
from jax.experimental.pallas import tpu as pltpu
from jax.experimental.pallas import tpu_sc as plsc

_N = 10000
_E = 320000
_H = 8
_D = 16
_HD = 128
_NP = 10240
_NW = 32
_EPW = _E // _NW
_BC = 80
_NCH = _EPW // _BC


def _node_proj_body(x_ref, wnt_ref, b1_ref, b2_ref, zn_ref, s1_ref, s2_ref):
    x = x_ref[...]
    zn = jnp.dot(x, wnt_ref[...], preferred_element_type=jnp.float32)
    zn_ref[...] = zn
    s1_ref[...] = jnp.dot(zn, b1_ref[...], preferred_element_type=jnp.float32)
    s2_ref[...] = jnp.dot(zn, b2_ref[...], preferred_element_type=jnp.float32)


def _node_proj(node_feats, WnT, B1p, B2p):
    bn = 2000
    grid = (_N // bn,)
    return pl.pallas_call(
        _node_proj_body,
        grid=grid,
        in_specs=[
            pl.BlockSpec((bn, 128), lambda i: (i, 0)),
            pl.BlockSpec((128, 128), lambda i: (0, 0)),
            pl.BlockSpec((128, 16), lambda i: (0, 0)),
            pl.BlockSpec((128, 16), lambda i: (0, 0)),
        ],
        out_specs=[
            pl.BlockSpec((bn, 128), lambda i: (i, 0)),
            pl.BlockSpec((bn, 16), lambda i: (i, 0)),
            pl.BlockSpec((bn, 16), lambda i: (i, 0)),
        ],
        out_shape=[
            jax.ShapeDtypeStruct((_N, 128), jnp.float32),
            jax.ShapeDtypeStruct((_N, 16), jnp.float32),
            jax.ShapeDtypeStruct((_N, 16), jnp.float32),
        ],
    )(node_feats, WnT, B1p, B2p)


def _edge_proj_body(x_ref, wet_ref, wse_ref, ze_ref, se_ref):
    x = x_ref[...]
    ze_ref[...] = jnp.dot(x, wet_ref[...], preferred_element_type=jnp.float32)
    se_ref[...] = jnp.dot(x, wse_ref[...], preferred_element_type=jnp.float32)


def _edge_proj(edge_feats, WeT, Wse):
    be = 8000
    grid = (_E // be,)
    return pl.pallas_call(
        _edge_proj_body,
        grid=grid,
        in_specs=[
            pl.BlockSpec((be, 16), lambda i: (i, 0)),
            pl.BlockSpec((16, 128), lambda i: (0, 0)),
            pl.BlockSpec((16, 16), lambda i: (0, 0)),
        ],
        out_specs=[
            pl.BlockSpec((be, 128), lambda i: (i, 0)),
            pl.BlockSpec((be, 16), lambda i: (i, 0)),
        ],
        out_shape=[
            jax.ShapeDtypeStruct((_E, 128), jnp.float32),
            jax.ShapeDtypeStruct((_E, 16), jnp.float32),
        ],
    )(edge_feats, WeT, Wse)


def _edge_pass_body(s1_hbm, s2_hbm, se_hbm, ze_hbm, src_hbm, dst_hbm,
                    cvec_hbm, zagg_hbm, zden_hbm,
                    aggp, denp,
                    src_v, dst_v, se_v, ze_v, g1_v, g2_v, al_v, wz_v, c_v,
                    agg_sh, den_sh, sem):
    core = lax.axis_index("c")
    sub = lax.axis_index("s")
    wid = sub * 2 + core
    base = sub * 640

    pltpu.sync_copy(zagg_hbm, agg_sh.at[pl.ds(base, 640)])
    pltpu.sync_copy(zden_hbm, den_sh.at[pl.ds(base, 640)])
    pltpu.sync_copy(cvec_hbm, c_v)
    plsc.subcore_barrier()

    e0 = wid * _EPW

    def chunk(i, carry):
        b = e0 + i * _BC
        pltpu.sync_copy(src_hbm.at[pl.ds(b, _BC)], src_v)
        pltpu.sync_copy(dst_hbm.at[pl.ds(b, _BC)], dst_v)
        pltpu.sync_copy(se_hbm.at[pl.ds(b, _BC)], se_v)
        pltpu.sync_copy(ze_hbm.at[pl.ds(b, _BC)], ze_v)
        pltpu.async_copy(s1_hbm.at[src_v], g1_v, sem).wait()
        pltpu.async_copy(s2_hbm.at[dst_v], g2_v, sem).wait()

        cv = c_v[...]

        def edge_body(j, c2):
            t = g1_v[j] + g2_v[j] + se_v[j]
            e = jnp.where(t > 0.0, t, t * jnp.float32(0.01))
            a = jnp.exp(e - cv)
            al_v[j] = a
            for h in range(_H):
                wz_v[j, pl.ds(h * 16, 16)] = ze_v[j, pl.ds(h * 16, 16)] * a[h]
            return c2

        lax.fori_loop(0, _BC, edge_body, 0, unroll=False)

        pltpu.sync_copy(wz_v, agg_sh.at[dst_v], add=True)
        pltpu.sync_copy(al_v, den_sh.at[dst_v], add=True)
        return carry

    lax.fori_loop(0, _NCH, chunk, 0, unroll=False)
    plsc.subcore_barrier()

    pltpu.sync_copy(agg_sh.at[pl.ds(base, 640)], aggp.at[core, pl.ds(base, 640)])
    pltpu.sync_copy(den_sh.at[pl.ds(base, 640)], denp.at[core, pl.ds(base, 640)])


def _edge_pass(S1, S2, se16, z_e, src, dst, cvec, zagg, zden):
    mesh = plsc.VectorSubcoreMesh(core_axis_name="c", subcore_axis_name="s")
    return pl.kernel(
        _edge_pass_body,
        out_type=[
            jax.ShapeDtypeStruct((2, _NP, 128), jnp.float32),
            jax.ShapeDtypeStruct((2, _NP, 16), jnp.float32),
        ],
        mesh=mesh,
        compiler_params=pltpu.CompilerParams(use_tc_tiling_on_sc=False),
        scratch_types=[
            pltpu.VMEM((_BC,), jnp.int32),
            pltpu.VMEM((_BC,), jnp.int32),
            pltpu.VMEM((_BC, 16), jnp.float32),
            pltpu.VMEM((_BC, 128), jnp.float32),
            pltpu.VMEM((_BC, 16), jnp.float32),
            pltpu.VMEM((_BC, 16), jnp.float32),
            pltpu.VMEM((_BC, 16), jnp.float32),
            pltpu.VMEM((_BC, 128), jnp.float32),
            pltpu.VMEM((16,), jnp.float32),
            pltpu.VMEM_SHARED((_NP, 128), jnp.float32),
            pltpu.VMEM_SHARED((_NP, 16), jnp.float32),
            pltpu.SemaphoreType.DMA,
        ],
    )(S1, S2, se16, z_e, src, dst, cvec, zagg, zden)


def _combine_body(aggp_ref, denp_ref, zn_ref, r_ref, wot_ref, out_ref):
    agg = aggp_ref[0] + aggp_ref[1]
    den = denp_ref[0] + denp_ref[1]
    denr = jnp.dot(den, r_ref[...], preferred_element_type=jnp.float32)
    mask = denr > 0.0
    den_safe = jnp.where(mask, denr, 1.0)
    z = jnp.where(mask, agg / den_safe, zn_ref[...])
    out = jnp.dot(z, wot_ref[...], preferred_element_type=jnp.float32)
    out_ref[...] = jnp.maximum(out, 0.0)


def _combine(aggp, denp, z_n, R, WoT):
    bn = 2000
    grid = (_N // bn,)
    return pl.pallas_call(
        _combine_body,
        grid=grid,
        in_specs=[
            pl.BlockSpec((2, bn, 128), lambda i: (0, i, 0)),
            pl.BlockSpec((2, bn, 16), lambda i: (0, i, 0)),
            pl.BlockSpec((bn, 128), lambda i: (i, 0)),
            pl.BlockSpec((16, 128), lambda i: (0, 0)),
            pl.BlockSpec((128, 16), lambda i: (0, 0)),
        ],
        out_specs=pl.BlockSpec((bn, 16), lambda i: (i, 0)),
        out_shape=jax.ShapeDtypeStruct((_N, 16), jnp.float32),
    )(aggp, denp, z_n, R, WoT)


def kernel(node_feats, edge_feats, edge_index, Wn, We, Wa, Wo):
    a1 = Wa[0, :_D]
    a2 = Wa[0, _D:]
    eye = jnp.eye(_H, dtype=jnp.float32)
    B1 = jnp.kron(eye, a1[:, None])
    B2 = jnp.kron(eye, a2[:, None])
    B12 = jnp.kron(eye, (a1 + a2)[:, None])
    pad = jnp.zeros((128, 8), jnp.float32)
    B1p = jnp.concatenate([B1, pad], axis=1)
    B2p = jnp.concatenate([B2, pad], axis=1)
    WeT = We.T
    Wse = WeT @ jnp.concatenate([B12, pad], axis=1)

    z_n, S1, S2 = _node_proj(node_feats, Wn.T, B1p, B2p)
    z_e, se16 = _edge_proj(edge_feats, WeT, Wse)

    c8 = jnp.maximum(jnp.max(S1[:, :_H], axis=0) + jnp.max(S2[:, :_H], axis=0)
                     + jnp.max(se16[:, :_H], axis=0), 0.0)
    cvec = jnp.concatenate([c8, jnp.full((8,), 1e30, jnp.float32)])

    zagg = jnp.zeros((640, 128), jnp.float32)
    zden = jnp.zeros((640, 16), jnp.float32)
    aggp, denp = _edge_pass(S1, S2, se16, z_e,
                            edge_index[0], edge_index[1], cvec, zagg, zden)

    R = jnp.kron(jnp.concatenate([eye, jnp.zeros((8, 8), jnp.float32)], axis=0),
                 jnp.ones((1, 16), jnp.float32))
    return _combine(aggp[:, :_N], denp[:, :_N], z_n, R, Wo.T)


# speedup vs baseline: 31.0965x; 1.2005x over previous
import functools
import jax
import jax.numpy as jnp
from jax import lax
from jax.experimental import pallas as pl
from jax.experimental.pallas import tpu as pltpu
from jax.experimental.pallas import tpu_sc as plsc

_N = 10000
_E = 320000
_H = 8
_D = 16
_HD = 128
_NP = 10240
_NW = 32
_EPW = _E // _NW
_BC = 40
_NCH = _EPW // _BC


def _node_proj_body(x_ref, wnt_ref, b1_ref, b2_ref, zn_ref, s1_ref, s2_ref):
    x = x_ref[...]
    zn = jnp.dot(x, wnt_ref[...], preferred_element_type=jnp.float32)
    zn_ref[...] = zn
    s1_ref[...] = jnp.dot(zn, b1_ref[...], preferred_element_type=jnp.float32)
    s2_ref[...] = jnp.dot(zn, b2_ref[...], preferred_element_type=jnp.float32)


def _node_proj(node_feats, WnT, B1p, B2p):
    bn = 2000
    grid = (_N // bn,)
    return pl.pallas_call(
        _node_proj_body,
        grid=grid,
        in_specs=[
            pl.BlockSpec((bn, 128), lambda i: (i, 0)),
            pl.BlockSpec((128, 128), lambda i: (0, 0)),
            pl.BlockSpec((128, 16), lambda i: (0, 0)),
            pl.BlockSpec((128, 16), lambda i: (0, 0)),
        ],
        out_specs=[
            pl.BlockSpec((bn, 128), lambda i: (i, 0)),
            pl.BlockSpec((bn, 16), lambda i: (i, 0)),
            pl.BlockSpec((bn, 16), lambda i: (i, 0)),
        ],
        out_shape=[
            jax.ShapeDtypeStruct((_N, 128), jnp.float32),
            jax.ShapeDtypeStruct((_N, 16), jnp.float32),
            jax.ShapeDtypeStruct((_N, 16), jnp.float32),
        ],
    )(node_feats, WnT, B1p, B2p)


def _edge_proj_body(x_ref, wet_ref, wse_ref, ze_ref, se_ref):
    x = x_ref[...]
    ze_ref[...] = jnp.dot(x, wet_ref[...], preferred_element_type=jnp.float32)
    se_ref[...] = jnp.dot(x, wse_ref[...], preferred_element_type=jnp.float32)


def _edge_proj(edge_feats, WeT, Wse):
    be = 8000
    grid = (_E // be,)
    return pl.pallas_call(
        _edge_proj_body,
        grid=grid,
        in_specs=[
            pl.BlockSpec((be, 16), lambda i: (i, 0)),
            pl.BlockSpec((16, 128), lambda i: (0, 0)),
            pl.BlockSpec((16, 16), lambda i: (0, 0)),
        ],
        out_specs=[
            pl.BlockSpec((be, 128), lambda i: (i, 0)),
            pl.BlockSpec((be, 16), lambda i: (i, 0)),
        ],
        out_shape=[
            jax.ShapeDtypeStruct((_E, 128), jnp.float32),
            jax.ShapeDtypeStruct((_E, 16), jnp.float32),
        ],
    )(edge_feats, WeT, Wse)


def _edge_pass_body(s1_hbm, s2_hbm, se_hbm, ze_hbm, src_hbm, dst_hbm,
                    cvec_hbm, zagg_hbm, zden_hbm,
                    aggp, denp,
                    src_v, dst_v, se_v, ze_v, g1_v, g2_v, al_v, wz_v, c_v,
                    agg_sh, den_sh,
                    sem_lin, sem_g1, sem_g2, sem_s1, sem_s2):
    core = lax.axis_index("c")
    sub = lax.axis_index("s")
    wid = sub * 2 + core
    base = sub * 640

    pltpu.sync_copy(zagg_hbm, agg_sh.at[pl.ds(base, 640)])
    pltpu.sync_copy(zden_hbm, den_sh.at[pl.ds(base, 640)])
    pltpu.sync_copy(cvec_hbm, c_v)
    plsc.subcore_barrier()

    e0 = wid * _EPW
    end = e0 + _EPW

    def lin_issue(i, p):
        @pl.when(i * _BC < _EPW)
        def _():
            b = e0 + i * _BC
            pltpu.async_copy(src_hbm.at[pl.ds(b, _BC)], src_v.at[p], sem_lin)
            pltpu.async_copy(dst_hbm.at[pl.ds(b, _BC)], dst_v.at[p], sem_lin)
            pltpu.async_copy(se_hbm.at[pl.ds(b, _BC)], se_v.at[p], sem_lin)
            pltpu.async_copy(ze_hbm.at[pl.ds(b, _BC)], ze_v.at[p], sem_lin)

    def lin_wait(p):
        pltpu.make_async_copy(src_hbm.at[pl.ds(0, _BC)], src_v.at[p], sem_lin).wait()
        pltpu.make_async_copy(dst_hbm.at[pl.ds(0, _BC)], dst_v.at[p], sem_lin).wait()
        pltpu.make_async_copy(se_hbm.at[pl.ds(0, _BC)], se_v.at[p], sem_lin).wait()
        pltpu.make_async_copy(ze_hbm.at[pl.ds(0, _BC)], ze_v.at[p], sem_lin).wait()

    def g_issue(p):
        pltpu.async_copy(s1_hbm.at[src_v.at[p]], g1_v.at[p], sem_g1)
        pltpu.async_copy(s2_hbm.at[dst_v.at[p]], g2_v.at[p], sem_g2)

    def g_wait(p):
        pltpu.make_async_copy(s1_hbm.at[src_v.at[p]], g1_v.at[p], sem_g1).wait()
        pltpu.make_async_copy(s2_hbm.at[dst_v.at[p]], g2_v.at[p], sem_g2).wait()

    def compute(p):
        cv = c_v[...]
        g1p, g2p, sep, zep = g1_v.at[p], g2_v.at[p], se_v.at[p], ze_v.at[p]
        alp, wzp = al_v.at[p], wz_v.at[p]

        def edge_body(j, c2):
            t = g1p[j] + g2p[j] + sep[j]
            e = jnp.where(t > 0.0, t, t * jnp.float32(0.01))
            a = jnp.exp(e - cv)
            alp[j] = a
            for h in range(_H):
                wzp[j, pl.ds(h * 16, 16)] = zep[j, pl.ds(h * 16, 16)] * a[h]
            return c2

        lax.fori_loop(0, _BC, edge_body, 0, unroll=False)

    def scat_issue(p):
        pltpu.async_copy(wz_v.at[p], agg_sh.at[dst_v.at[p]], sem_s1, add=True)
        pltpu.async_copy(al_v.at[p], den_sh.at[dst_v.at[p]], sem_s2, add=True)

    def scat_wait(p):
        pltpu.make_async_copy(wz_v.at[p], agg_sh.at[dst_v.at[p]], sem_s1).wait()
        pltpu.make_async_copy(al_v.at[p], den_sh.at[dst_v.at[p]], sem_s2).wait()

    lin_issue(0, 0)
    lin_wait(0)
    g_issue(0)
    lin_issue(1, 1)

    def step(i, p):
        g_wait(p)

        @pl.when(i + 1 < _NCH)
        def _():
            lin_wait(1 - p)
            g_issue(1 - p)

        compute(p)
        scat_issue(p)
        scat_wait(p)
        lin_issue(i + 2, p)

    def pair(k, carry):
        step(2 * k, 0)
        step(2 * k + 1, 1)
        return carry

    lax.fori_loop(0, _NCH // 2, pair, 0, unroll=False)
    plsc.subcore_barrier()

    pltpu.sync_copy(agg_sh.at[pl.ds(base, 640)], aggp.at[core, pl.ds(base, 640)])
    pltpu.sync_copy(den_sh.at[pl.ds(base, 640)], denp.at[core, pl.ds(base, 640)])


def _edge_pass(S1, S2, se16, z_e, src, dst, cvec, zagg, zden):
    mesh = plsc.VectorSubcoreMesh(core_axis_name="c", subcore_axis_name="s")
    return pl.kernel(
        _edge_pass_body,
        out_type=[
            jax.ShapeDtypeStruct((2, _NP, 128), jnp.float32),
            jax.ShapeDtypeStruct((2, _NP, 16), jnp.float32),
        ],
        mesh=mesh,
        compiler_params=pltpu.CompilerParams(use_tc_tiling_on_sc=False),
        scratch_types=[
            pltpu.VMEM((2, _BC), jnp.int32),
            pltpu.VMEM((2, _BC), jnp.int32),
            pltpu.VMEM((2, _BC, 16), jnp.float32),
            pltpu.VMEM((2, _BC, 128), jnp.float32),
            pltpu.VMEM((2, _BC, 16), jnp.float32),
            pltpu.VMEM((2, _BC, 16), jnp.float32),
            pltpu.VMEM((2, _BC, 16), jnp.float32),
            pltpu.VMEM((2, _BC, 128), jnp.float32),
            pltpu.VMEM((16,), jnp.float32),
            pltpu.VMEM_SHARED((_NP, 128), jnp.float32),
            pltpu.VMEM_SHARED((_NP, 16), jnp.float32),
            pltpu.SemaphoreType.DMA,
            pltpu.SemaphoreType.DMA,
            pltpu.SemaphoreType.DMA,
            pltpu.SemaphoreType.DMA,
            pltpu.SemaphoreType.DMA,
        ],
    )(S1, S2, se16, z_e, src, dst, cvec, zagg, zden)


def _combine_body(aggp_ref, denp_ref, zn_ref, r_ref, wot_ref, out_ref):
    agg = aggp_ref[0] + aggp_ref[1]
    den = denp_ref[0] + denp_ref[1]
    denr = jnp.dot(den, r_ref[...], preferred_element_type=jnp.float32)
    mask = denr > 0.0
    den_safe = jnp.where(mask, denr, 1.0)
    z = jnp.where(mask, agg / den_safe, zn_ref[...])
    out = jnp.dot(z, wot_ref[...], preferred_element_type=jnp.float32)
    out_ref[...] = jnp.maximum(out, 0.0)


def _combine(aggp, denp, z_n, R, WoT):
    bn = 2000
    grid = (_N // bn,)
    return pl.pallas_call(
        _combine_body,
        grid=grid,
        in_specs=[
            pl.BlockSpec((2, bn, 128), lambda i: (0, i, 0)),
            pl.BlockSpec((2, bn, 16), lambda i: (0, i, 0)),
            pl.BlockSpec((bn, 128), lambda i: (i, 0)),
            pl.BlockSpec((16, 128), lambda i: (0, 0)),
            pl.BlockSpec((128, 16), lambda i: (0, 0)),
        ],
        out_specs=pl.BlockSpec((bn, 16), lambda i: (i, 0)),
        out_shape=jax.ShapeDtypeStruct((_N, 16), jnp.float32),
    )(aggp, denp, z_n, R, WoT)


def kernel(node_feats, edge_feats, edge_index, Wn, We, Wa, Wo):
    a1 = Wa[0, :_D]
    a2 = Wa[0, _D:]
    eye = jnp.eye(_H, dtype=jnp.float32)
    B1 = jnp.kron(eye, a1[:, None])
    B2 = jnp.kron(eye, a2[:, None])
    B12 = jnp.kron(eye, (a1 + a2)[:, None])
    pad = jnp.zeros((128, 8), jnp.float32)
    B1p = jnp.concatenate([B1, pad], axis=1)
    B2p = jnp.concatenate([B2, pad], axis=1)
    WeT = We.T
    Wse = WeT @ jnp.concatenate([B12, pad], axis=1)

    z_n, S1, S2 = _node_proj(node_feats, Wn.T, B1p, B2p)
    z_e, se16 = _edge_proj(edge_feats, WeT, Wse)

    c8 = jnp.maximum(jnp.max(S1[:, :_H], axis=0) + jnp.max(S2[:, :_H], axis=0)
                     + jnp.max(se16[:, :_H], axis=0), 0.0)
    cvec = jnp.concatenate([c8, jnp.full((8,), 1e30, jnp.float32)])

    zagg = jnp.zeros((640, 128), jnp.float32)
    zden = jnp.zeros((640, 16), jnp.float32)
    aggp, denp = _edge_pass(S1, S2, se16, z_e,
                            edge_index[0], edge_index[1], cvec, zagg, zden)

    R = jnp.kron(jnp.concatenate([eye, jnp.zeros((8, 8), jnp.float32)], axis=0),
                 jnp.ones((1, 16), jnp.float32))
    return _combine(aggp[:, :_N], denp[:, :_N], z_n, R, Wo.T)


# speedup vs baseline: 32.7693x; 1.0538x over previous
import functools
import jax
import jax.numpy as jnp
from jax import lax
from jax.experimental import pallas as pl
from jax.experimental.pallas import tpu as pltpu
from jax.experimental.pallas import tpu_sc as plsc

_N = 10000
_E = 320000
_H = 8
_D = 16
_HD = 128
_NP = 10240
_NW = 32
_EPW = _E // _NW
_BC = 40
_NCH = _EPW // _BC


def _node_proj_body(x_ref, wnt_ref, b1_ref, b2_ref, zn_ref, s1_ref, s2_ref):
    x = x_ref[...]
    zn = jnp.dot(x, wnt_ref[...], preferred_element_type=jnp.float32)
    zn_ref[...] = zn
    s1_ref[...] = jnp.dot(zn, b1_ref[...], preferred_element_type=jnp.float32)
    s2_ref[...] = jnp.dot(zn, b2_ref[...], preferred_element_type=jnp.float32)


def _node_proj(node_feats, WnT, B1p, B2p):
    bn = 2000
    grid = (_N // bn,)
    return pl.pallas_call(
        _node_proj_body,
        grid=grid,
        in_specs=[
            pl.BlockSpec((bn, 128), lambda i: (i, 0)),
            pl.BlockSpec((128, 128), lambda i: (0, 0)),
            pl.BlockSpec((128, 16), lambda i: (0, 0)),
            pl.BlockSpec((128, 16), lambda i: (0, 0)),
        ],
        out_specs=[
            pl.BlockSpec((bn, 128), lambda i: (i, 0)),
            pl.BlockSpec((bn, 16), lambda i: (i, 0)),
            pl.BlockSpec((bn, 16), lambda i: (i, 0)),
        ],
        out_shape=[
            jax.ShapeDtypeStruct((_N, 128), jnp.float32),
            jax.ShapeDtypeStruct((_N, 16), jnp.float32),
            jax.ShapeDtypeStruct((_N, 16), jnp.float32),
        ],
    )(node_feats, WnT, B1p, B2p)


def _edge_proj_body(x_ref, wet_ref, wse_ref, ze_ref, se_ref):
    x = x_ref[...]
    ze_ref[...] = jnp.dot(x, wet_ref[...], preferred_element_type=jnp.float32)
    se_ref[...] = jnp.dot(x, wse_ref[...], preferred_element_type=jnp.float32)


def _edge_proj(edge_feats, WeT, Wse):
    be = 8000
    grid = (_E // be,)
    return pl.pallas_call(
        _edge_proj_body,
        grid=grid,
        in_specs=[
            pl.BlockSpec((be, 16), lambda i: (i, 0)),
            pl.BlockSpec((16, 128), lambda i: (0, 0)),
            pl.BlockSpec((16, 16), lambda i: (0, 0)),
        ],
        out_specs=[
            pl.BlockSpec((be, 128), lambda i: (i, 0)),
            pl.BlockSpec((be, 16), lambda i: (i, 0)),
        ],
        out_shape=[
            jax.ShapeDtypeStruct((_E, 128), jnp.float32),
            jax.ShapeDtypeStruct((_E, 16), jnp.float32),
        ],
    )(edge_feats, WeT, Wse)


def _edge_pass_body(s1_hbm, s2_hbm, se_hbm, ze_hbm, src_hbm, dst_hbm,
                    cvec_hbm, zagg_hbm, zden_hbm,
                    aggp, denp,
                    src_v, dst_v, dsc_v, se_v, ze_v, g1_v, g2_v, al_v, wz_v,
                    c_v, agg_sh, den_sh,
                    sem_lin, sem_g1, sem_g2, sem_s1, sem_s2):
    core = lax.axis_index("c")
    sub = lax.axis_index("s")
    wid = sub * 2 + core
    base = sub * 640

    pltpu.sync_copy(zagg_hbm, agg_sh.at[pl.ds(base, 640)])
    pltpu.sync_copy(zden_hbm, den_sh.at[pl.ds(base, 640)])
    pltpu.sync_copy(cvec_hbm, c_v)
    plsc.subcore_barrier()

    e0 = wid * _EPW
    end = e0 + _EPW

    def lin_issue(i, p):
        @pl.when(i * _BC < _EPW)
        def _():
            b = e0 + i * _BC
            pltpu.async_copy(src_hbm.at[pl.ds(b, _BC)], src_v.at[p], sem_lin)
            pltpu.async_copy(dst_hbm.at[pl.ds(b, _BC)], dst_v.at[p], sem_lin)
            pltpu.async_copy(se_hbm.at[pl.ds(b, _BC)], se_v.at[p], sem_lin)
            pltpu.async_copy(ze_hbm.at[pl.ds(b, _BC)], ze_v.at[p], sem_lin)

    def lin_wait(p):
        pltpu.make_async_copy(src_hbm.at[pl.ds(0, _BC)], src_v.at[p], sem_lin).wait()
        pltpu.make_async_copy(dst_hbm.at[pl.ds(0, _BC)], dst_v.at[p], sem_lin).wait()
        pltpu.make_async_copy(se_hbm.at[pl.ds(0, _BC)], se_v.at[p], sem_lin).wait()
        pltpu.make_async_copy(ze_hbm.at[pl.ds(0, _BC)], ze_v.at[p], sem_lin).wait()

    def g_issue(p):
        pltpu.async_copy(s1_hbm.at[src_v.at[p]], g1_v.at[p], sem_g1)
        pltpu.async_copy(s2_hbm.at[dst_v.at[p]], g2_v.at[p], sem_g2)

    def g_wait(p):
        pltpu.make_async_copy(s1_hbm.at[src_v.at[p]], g1_v.at[p], sem_g1).wait()
        pltpu.make_async_copy(s2_hbm.at[dst_v.at[p]], g2_v.at[p], sem_g2).wait()

    def compute(p):
        cv = c_v[...]
        g1p, g2p, sep, zep = g1_v.at[p], g2_v.at[p], se_v.at[p], ze_v.at[p]
        alp, wzp = al_v.at[p], wz_v.at[p]

        def edge_body(j, c2):
            t = g1p[j] + g2p[j] + sep[j]
            e = jnp.where(t > 0.0, t, t * jnp.float32(0.01))
            a = jnp.exp(e - cv)
            alp[j] = a
            for h in range(_H):
                wzp[j, pl.ds(h * 16, 16)] = zep[j, pl.ds(h * 16, 16)] * a[h]
            return c2

        lax.fori_loop(0, _BC, edge_body, 0, unroll=False)

    def scat_issue(p):
        pltpu.async_copy(wz_v.at[p], agg_sh.at[dsc_v.at[p]], sem_s1, add=True)
        pltpu.async_copy(al_v.at[p], den_sh.at[dsc_v.at[p]], sem_s2, add=True)

    def scat_wait(p):
        pltpu.make_async_copy(wz_v.at[p], agg_sh.at[dsc_v.at[p]], sem_s1).wait()
        pltpu.make_async_copy(al_v.at[p], den_sh.at[dsc_v.at[p]], sem_s2).wait()

    lin_issue(0, 0)
    lin_wait(0)
    g_issue(0)
    lin_issue(1, 1)

    def step(i, p):
        g_wait(p)

        @pl.when(i + 1 < _NCH)
        def _():
            lin_wait(1 - p)
            g_issue(1 - p)

        @pl.when(i >= 2)
        def _():
            scat_wait(p)

        for o in (0, 16, 24):
            dsc_v.at[p][pl.ds(o, 16)] = dst_v.at[p][pl.ds(o, 16)]
        compute(p)
        scat_issue(p)
        lin_issue(i + 2, p)

    def pair(k, carry):
        step(2 * k, 0)
        step(2 * k + 1, 1)
        return carry

    lax.fori_loop(0, _NCH // 2, pair, 0, unroll=False)
    scat_wait(0)
    scat_wait(1)
    plsc.subcore_barrier()

    pltpu.sync_copy(agg_sh.at[pl.ds(base, 640)], aggp.at[core, pl.ds(base, 640)])
    pltpu.sync_copy(den_sh.at[pl.ds(base, 640)], denp.at[core, pl.ds(base, 640)])


def _edge_pass(S1, S2, se16, z_e, src, dst, cvec, zagg, zden):
    mesh = plsc.VectorSubcoreMesh(core_axis_name="c", subcore_axis_name="s")
    return pl.kernel(
        _edge_pass_body,
        out_type=[
            jax.ShapeDtypeStruct((2, _NP, 128), jnp.float32),
            jax.ShapeDtypeStruct((2, _NP, 16), jnp.float32),
        ],
        mesh=mesh,
        compiler_params=pltpu.CompilerParams(use_tc_tiling_on_sc=False),
        scratch_types=[
            pltpu.VMEM((2, _BC), jnp.int32),
            pltpu.VMEM((2, _BC), jnp.int32),
            pltpu.VMEM((2, _BC), jnp.int32),
            pltpu.VMEM((2, _BC, 16), jnp.float32),
            pltpu.VMEM((2, _BC, 128), jnp.float32),
            pltpu.VMEM((2, _BC, 16), jnp.float32),
            pltpu.VMEM((2, _BC, 16), jnp.float32),
            pltpu.VMEM((2, _BC, 16), jnp.float32),
            pltpu.VMEM((2, _BC, 128), jnp.float32),
            pltpu.VMEM((16,), jnp.float32),
            pltpu.VMEM_SHARED((_NP, 128), jnp.float32),
            pltpu.VMEM_SHARED((_NP, 16), jnp.float32),
            pltpu.SemaphoreType.DMA,
            pltpu.SemaphoreType.DMA,
            pltpu.SemaphoreType.DMA,
            pltpu.SemaphoreType.DMA,
            pltpu.SemaphoreType.DMA,
        ],
    )(S1, S2, se16, z_e, src, dst, cvec, zagg, zden)


def _combine_body(aggp_ref, denp_ref, zn_ref, r_ref, wot_ref, out_ref):
    agg = aggp_ref[0] + aggp_ref[1]
    den = denp_ref[0] + denp_ref[1]
    denr = jnp.dot(den, r_ref[...], preferred_element_type=jnp.float32)
    mask = denr > 0.0
    den_safe = jnp.where(mask, denr, 1.0)
    z = jnp.where(mask, agg / den_safe, zn_ref[...])
    out = jnp.dot(z, wot_ref[...], preferred_element_type=jnp.float32)
    out_ref[...] = jnp.maximum(out, 0.0)


def _combine(aggp, denp, z_n, R, WoT):
    bn = 2000
    grid = (_N // bn,)
    return pl.pallas_call(
        _combine_body,
        grid=grid,
        in_specs=[
            pl.BlockSpec((2, bn, 128), lambda i: (0, i, 0)),
            pl.BlockSpec((2, bn, 16), lambda i: (0, i, 0)),
            pl.BlockSpec((bn, 128), lambda i: (i, 0)),
            pl.BlockSpec((16, 128), lambda i: (0, 0)),
            pl.BlockSpec((128, 16), lambda i: (0, 0)),
        ],
        out_specs=pl.BlockSpec((bn, 16), lambda i: (i, 0)),
        out_shape=jax.ShapeDtypeStruct((_N, 16), jnp.float32),
    )(aggp, denp, z_n, R, WoT)


def kernel(node_feats, edge_feats, edge_index, Wn, We, Wa, Wo):
    a1 = Wa[0, :_D]
    a2 = Wa[0, _D:]
    eye = jnp.eye(_H, dtype=jnp.float32)
    B1 = jnp.kron(eye, a1[:, None])
    B2 = jnp.kron(eye, a2[:, None])
    B12 = jnp.kron(eye, (a1 + a2)[:, None])
    pad = jnp.zeros((128, 8), jnp.float32)
    B1p = jnp.concatenate([B1, pad], axis=1)
    B2p = jnp.concatenate([B2, pad], axis=1)
    WeT = We.T
    Wse = WeT @ jnp.concatenate([B12, pad], axis=1)

    z_n, S1, S2 = _node_proj(node_feats, Wn.T, B1p, B2p)
    z_e, se16 = _edge_proj(edge_feats, WeT, Wse)

    c8 = jnp.maximum(jnp.max(S1[:, :_H], axis=0) + jnp.max(S2[:, :_H], axis=0)
                     + jnp.max(se16[:, :_H], axis=0), 0.0)
    cvec = jnp.concatenate([c8, jnp.full((8,), 1e30, jnp.float32)])

    zagg = jnp.zeros((640, 128), jnp.float32)
    zden = jnp.zeros((640, 16), jnp.float32)
    aggp, denp = _edge_pass(S1, S2, se16, z_e,
                            edge_index[0], edge_index[1], cvec, zagg, zden)

    R = jnp.kron(jnp.concatenate([eye, jnp.zeros((8, 8), jnp.float32)], axis=0),
                 jnp.ones((1, 16), jnp.float32))
    return _combine(aggp[:, :_N], denp[:, :_N], z_n, R, Wo.T)


# speedup vs baseline: 43.7438x; 1.3349x over previous
import functools
import jax
import jax.numpy as jnp
from jax import lax
from jax.experimental import pallas as pl
from jax.experimental.pallas import tpu as pltpu
from jax.experimental.pallas import tpu_sc as plsc

_N = 10000
_E = 320000
_H = 8
_D = 16
_HD = 128
_NP = 10240
_NW = 32
_EPW = _E // _NW
_BC = 40
_NCH = _EPW // _BC


def _node_proj_body(x_ref, wnt_ref, b1_ref, b2_ref, zn_ref, s1_ref, s2_ref):
    x = x_ref[...]
    zn = jnp.dot(x, wnt_ref[...], preferred_element_type=jnp.float32)
    zn_ref[...] = zn
    s1_ref[...] = jnp.dot(zn, b1_ref[...], preferred_element_type=jnp.float32)
    s2_ref[...] = jnp.dot(zn, b2_ref[...], preferred_element_type=jnp.float32)


def _node_proj(node_feats, WnT, B1p, B2p):
    bn = 2000
    grid = (_N // bn,)
    return pl.pallas_call(
        _node_proj_body,
        grid=grid,
        in_specs=[
            pl.BlockSpec((bn, 128), lambda i: (i, 0)),
            pl.BlockSpec((128, 128), lambda i: (0, 0)),
            pl.BlockSpec((128, 16), lambda i: (0, 0)),
            pl.BlockSpec((128, 16), lambda i: (0, 0)),
        ],
        out_specs=[
            pl.BlockSpec((bn, 128), lambda i: (i, 0)),
            pl.BlockSpec((bn, 16), lambda i: (i, 0)),
            pl.BlockSpec((bn, 16), lambda i: (i, 0)),
        ],
        out_shape=[
            jax.ShapeDtypeStruct((_N, 128), jnp.float32),
            jax.ShapeDtypeStruct((_N, 16), jnp.float32),
            jax.ShapeDtypeStruct((_N, 16), jnp.float32),
        ],
    )(node_feats, WnT, B1p, B2p)


def _edge_proj_body(x_ref, wse_ref, se_ref):
    se_ref[...] = jnp.dot(x_ref[...], wse_ref[...],
                          preferred_element_type=jnp.float32)


def _edge_proj(edge_feats, Wse):
    be = 8000
    grid = (_E // be,)
    return pl.pallas_call(
        _edge_proj_body,
        grid=grid,
        in_specs=[
            pl.BlockSpec((be, 16), lambda i: (i, 0)),
            pl.BlockSpec((16, 16), lambda i: (0, 0)),
        ],
        out_specs=pl.BlockSpec((be, 16), lambda i: (i, 0)),
        out_shape=jax.ShapeDtypeStruct((_E, 16), jnp.float32),
    )(edge_feats, Wse)


def _edge_pass_body(s1_hbm, s2_hbm, se_hbm, ef_hbm, src_hbm, dst_hbm,
                    cvec_hbm, zagg_hbm, zden_hbm,
                    aggp, denp,
                    src_v, dst_v, dsc_v, se_v, ze_v, g1_v, g2_v, al_v, wz_v,
                    c_v, agg_sh, den_sh,
                    sem_lin, sem_g1, sem_g2, sem_s1, sem_s2):
    core = lax.axis_index("c")
    sub = lax.axis_index("s")
    wid = sub * 2 + core
    base = sub * 640

    pltpu.sync_copy(zagg_hbm, agg_sh.at[pl.ds(base, 640)])
    pltpu.sync_copy(zden_hbm, den_sh.at[pl.ds(base, 640)])
    pltpu.sync_copy(cvec_hbm, c_v)
    plsc.subcore_barrier()

    e0 = wid * _EPW
    end = e0 + _EPW

    def lin_issue(i, p):
        @pl.when(i * _BC < _EPW)
        def _():
            b = e0 + i * _BC
            pltpu.async_copy(src_hbm.at[pl.ds(b, _BC)], src_v.at[p], sem_lin)
            pltpu.async_copy(dst_hbm.at[pl.ds(b, _BC)], dst_v.at[p], sem_lin)
            pltpu.async_copy(se_hbm.at[pl.ds(b, _BC)], se_v.at[p], sem_lin)
            pltpu.async_copy(ef_hbm.at[pl.ds(b, _BC)], ze_v.at[p], sem_lin)

    def lin_wait(p):
        pltpu.make_async_copy(src_hbm.at[pl.ds(0, _BC)], src_v.at[p], sem_lin).wait()
        pltpu.make_async_copy(dst_hbm.at[pl.ds(0, _BC)], dst_v.at[p], sem_lin).wait()
        pltpu.make_async_copy(se_hbm.at[pl.ds(0, _BC)], se_v.at[p], sem_lin).wait()
        pltpu.make_async_copy(ef_hbm.at[pl.ds(0, _BC)], ze_v.at[p], sem_lin).wait()

    def g_issue(p):
        pltpu.async_copy(s1_hbm.at[src_v.at[p]], g1_v.at[p], sem_g1)
        pltpu.async_copy(s2_hbm.at[dst_v.at[p]], g2_v.at[p], sem_g2)

    def g_wait(p):
        pltpu.make_async_copy(s1_hbm.at[src_v.at[p]], g1_v.at[p], sem_g1).wait()
        pltpu.make_async_copy(s2_hbm.at[dst_v.at[p]], g2_v.at[p], sem_g2).wait()

    def compute(p):
        cv = c_v[...]
        g1p, g2p, sep, zep = g1_v.at[p], g2_v.at[p], se_v.at[p], ze_v.at[p]
        alp, wzp = al_v.at[p], wz_v.at[p]

        def edge_body(k, c2):
            for u in range(2):
                j = k * 2 + u
                t = g1p[j] + g2p[j] + sep[j]
                e = jnp.where(t > 0.0, t, t * jnp.float32(0.01))
                a = jnp.exp(e - cv)
                alp[j] = a
                ef = zep[j]
                for h in range(_H):
                    wzp[j, pl.ds(h * 16, 16)] = ef * a[h]
            return c2

        lax.fori_loop(0, _BC // 2, edge_body, 0, unroll=False)

    def scat_issue(p):
        pltpu.async_copy(wz_v.at[p], agg_sh.at[dsc_v.at[p]], sem_s1, add=True)
        pltpu.async_copy(al_v.at[p], den_sh.at[dsc_v.at[p]], sem_s2, add=True)

    def scat_wait(p):
        pltpu.make_async_copy(wz_v.at[p], agg_sh.at[dsc_v.at[p]], sem_s1).wait()
        pltpu.make_async_copy(al_v.at[p], den_sh.at[dsc_v.at[p]], sem_s2).wait()

    lin_issue(0, 0)
    lin_wait(0)
    g_issue(0)
    lin_issue(1, 1)

    def step(i, p):
        g_wait(p)

        @pl.when(i + 1 < _NCH)
        def _():
            lin_wait(1 - p)
            g_issue(1 - p)

        @pl.when(i >= 2)
        def _():
            scat_wait(p)

        for o in (0, 16, 24):
            dsc_v.at[p][pl.ds(o, 16)] = dst_v.at[p][pl.ds(o, 16)]
        compute(p)
        scat_issue(p)
        lin_issue(i + 2, p)

    def pair(k, carry):
        step(2 * k, 0)
        step(2 * k + 1, 1)
        return carry

    lax.fori_loop(0, _NCH // 2, pair, 0, unroll=False)
    scat_wait(0)
    scat_wait(1)
    plsc.subcore_barrier()

    pltpu.sync_copy(agg_sh.at[pl.ds(base, 640)], aggp.at[core, pl.ds(base, 640)])
    pltpu.sync_copy(den_sh.at[pl.ds(base, 640)], denp.at[core, pl.ds(base, 640)])


def _edge_pass(S1, S2, se16, edge_feats, src, dst, cvec, zagg, zden):
    mesh = plsc.VectorSubcoreMesh(core_axis_name="c", subcore_axis_name="s")
    return pl.kernel(
        _edge_pass_body,
        out_type=[
            jax.ShapeDtypeStruct((2, _NP, 128), jnp.float32),
            jax.ShapeDtypeStruct((2, _NP, 16), jnp.float32),
        ],
        mesh=mesh,
        compiler_params=pltpu.CompilerParams(use_tc_tiling_on_sc=False),
        scratch_types=[
            pltpu.VMEM((2, _BC), jnp.int32),
            pltpu.VMEM((2, _BC), jnp.int32),
            pltpu.VMEM((2, _BC), jnp.int32),
            pltpu.VMEM((2, _BC, 16), jnp.float32),
            pltpu.VMEM((2, _BC, 16), jnp.float32),
            pltpu.VMEM((2, _BC, 16), jnp.float32),
            pltpu.VMEM((2, _BC, 16), jnp.float32),
            pltpu.VMEM((2, _BC, 16), jnp.float32),
            pltpu.VMEM((2, _BC, 128), jnp.float32),
            pltpu.VMEM((16,), jnp.float32),
            pltpu.VMEM_SHARED((_NP, 128), jnp.float32),
            pltpu.VMEM_SHARED((_NP, 16), jnp.float32),
            pltpu.SemaphoreType.DMA,
            pltpu.SemaphoreType.DMA,
            pltpu.SemaphoreType.DMA,
            pltpu.SemaphoreType.DMA,
            pltpu.SemaphoreType.DMA,
        ],
    )(S1, S2, se16, edge_feats, src, dst, cvec, zagg, zden)


def _combine_body(aggp_ref, denp_ref, zn_ref, r_ref, bd_ref, wot_ref, out_ref):
    agg = aggp_ref[0] + aggp_ref[1]
    den = denp_ref[0] + denp_ref[1]
    denr = jnp.dot(den, r_ref[...], preferred_element_type=jnp.float32)
    mask = denr > 0.0
    den_safe = jnp.where(mask, denr, 1.0)
    zz = jnp.dot(agg / den_safe, bd_ref[...], preferred_element_type=jnp.float32)
    z = jnp.where(mask, zz, zn_ref[...])
    out = jnp.dot(z, wot_ref[...], preferred_element_type=jnp.float32)
    out_ref[...] = jnp.maximum(out, 0.0)


def _combine(aggp, denp, z_n, R, BD, WoT):
    bn = 2000
    grid = (_N // bn,)
    return pl.pallas_call(
        _combine_body,
        grid=grid,
        in_specs=[
            pl.BlockSpec((2, bn, 128), lambda i: (0, i, 0)),
            pl.BlockSpec((2, bn, 16), lambda i: (0, i, 0)),
            pl.BlockSpec((bn, 128), lambda i: (i, 0)),
            pl.BlockSpec((16, 128), lambda i: (0, 0)),
            pl.BlockSpec((128, 128), lambda i: (0, 0)),
            pl.BlockSpec((128, 16), lambda i: (0, 0)),
        ],
        out_specs=pl.BlockSpec((bn, 16), lambda i: (i, 0)),
        out_shape=jax.ShapeDtypeStruct((_N, 16), jnp.float32),
    )(aggp, denp, z_n, R, BD, WoT)


def kernel(node_feats, edge_feats, edge_index, Wn, We, Wa, Wo):
    a1 = Wa[0, :_D]
    a2 = Wa[0, _D:]
    eye = jnp.eye(_H, dtype=jnp.float32)
    B1 = jnp.kron(eye, a1[:, None])
    B2 = jnp.kron(eye, a2[:, None])
    B12 = jnp.kron(eye, (a1 + a2)[:, None])
    pad = jnp.zeros((128, 8), jnp.float32)
    B1p = jnp.concatenate([B1, pad], axis=1)
    B2p = jnp.concatenate([B2, pad], axis=1)
    WeT = We.T
    Wse = WeT @ jnp.concatenate([B12, pad], axis=1)

    z_n, S1, S2 = _node_proj(node_feats, Wn.T, B1p, B2p)
    se16 = _edge_proj(edge_feats, Wse)

    c8 = jnp.maximum(jnp.max(S1[:, :_H], axis=0) + jnp.max(S2[:, :_H], axis=0)
                     + jnp.max(se16[:, :_H], axis=0), 0.0)
    cvec = jnp.concatenate([c8, jnp.full((8,), 1e30, jnp.float32)])

    zagg = jnp.zeros((640, 128), jnp.float32)
    zden = jnp.zeros((640, 16), jnp.float32)
    aggp, denp = _edge_pass(S1, S2, se16, edge_feats,
                            edge_index[0], edge_index[1], cvec, zagg, zden)

    R = jnp.kron(jnp.concatenate([eye, jnp.zeros((8, 8), jnp.float32)], axis=0),
                 jnp.ones((1, 16), jnp.float32))
    blockmask = jnp.kron(eye, jnp.ones((16, 16), jnp.float32))
    BD = jnp.tile(WeT, (_H, 1)) * blockmask
    return _combine(aggp[:, :_N], denp[:, :_N], z_n, R, BD, Wo.T)


# speedup vs baseline: 47.0992x; 1.0767x over previous
import functools
import jax
import jax.numpy as jnp
from jax import lax
from jax.experimental import pallas as pl
from jax.experimental.pallas import tpu as pltpu
from jax.experimental.pallas import tpu_sc as plsc

_N = 10000
_E = 320000
_H = 8
_D = 16
_HD = 128
_NP = 10240
_NW = 32
_EPW = _E // _NW
_BC = 80
_NCH = _EPW // _BC


def _node_proj_body(x_ref, wnt_ref, b1_ref, b2_ref, zn_ref, s1_ref, s2_ref):
    x = x_ref[...]
    zn = jnp.dot(x, wnt_ref[...], preferred_element_type=jnp.float32)
    zn_ref[...] = zn
    s1_ref[...] = jnp.dot(zn, b1_ref[...], preferred_element_type=jnp.float32)
    s2_ref[...] = jnp.dot(zn, b2_ref[...], preferred_element_type=jnp.float32)


def _node_proj(node_feats, WnT, B1p, B2p):
    bn = 2000
    grid = (_N // bn,)
    return pl.pallas_call(
        _node_proj_body,
        grid=grid,
        in_specs=[
            pl.BlockSpec((bn, 128), lambda i: (i, 0)),
            pl.BlockSpec((128, 128), lambda i: (0, 0)),
            pl.BlockSpec((128, 16), lambda i: (0, 0)),
            pl.BlockSpec((128, 16), lambda i: (0, 0)),
        ],
        out_specs=[
            pl.BlockSpec((bn, 128), lambda i: (i, 0)),
            pl.BlockSpec((bn, 16), lambda i: (i, 0)),
            pl.BlockSpec((bn, 16), lambda i: (i, 0)),
        ],
        out_shape=[
            jax.ShapeDtypeStruct((_N, 128), jnp.float32),
            jax.ShapeDtypeStruct((_N, 16), jnp.float32),
            jax.ShapeDtypeStruct((_N, 16), jnp.float32),
        ],
    )(node_feats, WnT, B1p, B2p)


def _edge_proj_body(x_ref, wse_ref, se_ref):
    se_ref[...] = jnp.dot(x_ref[...], wse_ref[...],
                          preferred_element_type=jnp.float32)


def _edge_proj(edge_feats, Wse):
    be = 8000
    grid = (_E // be,)
    return pl.pallas_call(
        _edge_proj_body,
        grid=grid,
        in_specs=[
            pl.BlockSpec((be, 16), lambda i: (i, 0)),
            pl.BlockSpec((16, 16), lambda i: (0, 0)),
        ],
        out_specs=pl.BlockSpec((be, 16), lambda i: (i, 0)),
        out_shape=jax.ShapeDtypeStruct((_E, 16), jnp.float32),
    )(edge_feats, Wse)


def _edge_pass_body(s1_hbm, s2_hbm, se_hbm, ef_hbm, src_hbm, dst_hbm,
                    cvec_hbm, zagg_hbm, zden_hbm,
                    aggp, denp,
                    src_v, dst_v, dsc_v, se_v, ze_v, g1_v, g2_v, al_v, wz_v,
                    c_v, agg_sh, den_sh,
                    sem_lin, sem_g1, sem_g2, sem_s1, sem_s2):
    core = lax.axis_index("c")
    sub = lax.axis_index("s")
    wid = sub * 2 + core
    base = sub * 640

    pltpu.sync_copy(zagg_hbm, agg_sh.at[pl.ds(base, 640)])
    pltpu.sync_copy(zden_hbm, den_sh.at[pl.ds(base, 640)])
    pltpu.sync_copy(cvec_hbm, c_v)
    plsc.subcore_barrier()

    e0 = wid * _EPW
    end = e0 + _EPW

    def lin_issue(i, p):
        @pl.when(i * _BC < _EPW)
        def _():
            b = e0 + i * _BC
            pltpu.async_copy(src_hbm.at[pl.ds(b, _BC)], src_v.at[p], sem_lin)
            pltpu.async_copy(dst_hbm.at[pl.ds(b, _BC)], dst_v.at[p], sem_lin)
            pltpu.async_copy(se_hbm.at[pl.ds(b, _BC)], se_v.at[p], sem_lin)
            pltpu.async_copy(ef_hbm.at[pl.ds(b, _BC)], ze_v.at[p], sem_lin)

    def lin_wait(p):
        pltpu.make_async_copy(src_hbm.at[pl.ds(0, _BC)], src_v.at[p], sem_lin).wait()
        pltpu.make_async_copy(dst_hbm.at[pl.ds(0, _BC)], dst_v.at[p], sem_lin).wait()
        pltpu.make_async_copy(se_hbm.at[pl.ds(0, _BC)], se_v.at[p], sem_lin).wait()
        pltpu.make_async_copy(ef_hbm.at[pl.ds(0, _BC)], ze_v.at[p], sem_lin).wait()

    def g_issue(p):
        pltpu.async_copy(s1_hbm.at[src_v.at[p]], g1_v.at[p], sem_g1)
        pltpu.async_copy(s2_hbm.at[dst_v.at[p]], g2_v.at[p], sem_g2)

    def g_wait(p):
        pltpu.make_async_copy(s1_hbm.at[src_v.at[p]], g1_v.at[p], sem_g1).wait()
        pltpu.make_async_copy(s2_hbm.at[dst_v.at[p]], g2_v.at[p], sem_g2).wait()

    def compute(p):
        cv = c_v[...]
        g1p, g2p, sep, zep = g1_v.at[p], g2_v.at[p], se_v.at[p], ze_v.at[p]
        alp, wzp = al_v.at[p], wz_v.at[p]

        def edge_body(k, c2):
            for u in range(2):
                j = k * 2 + u
                t = g1p[j] + g2p[j] + sep[j]
                e = jnp.where(t > 0.0, t, t * jnp.float32(0.01))
                a = jnp.exp(e - cv)
                alp[j] = a
                ef = zep[j]
                for h in range(_H):
                    wzp[j, pl.ds(h * 16, 16)] = ef * a[h]
            return c2

        lax.fori_loop(0, _BC // 2, edge_body, 0, unroll=False)

    def scat_issue(p):
        pltpu.async_copy(wz_v.at[p], agg_sh.at[dsc_v.at[p]], sem_s1, add=True)
        pltpu.async_copy(al_v.at[p], den_sh.at[dsc_v.at[p]], sem_s2, add=True)

    def scat_wait(p):
        pltpu.make_async_copy(wz_v.at[p], agg_sh.at[dsc_v.at[p]], sem_s1).wait()
        pltpu.make_async_copy(al_v.at[p], den_sh.at[dsc_v.at[p]], sem_s2).wait()

    lin_issue(0, 0)
    lin_wait(0)
    g_issue(0)
    lin_issue(1, 1)

    def step(i, p):
        g_wait(p)

        @pl.when(i + 1 < _NCH)
        def _():
            lin_wait(1 - p)
            g_issue(1 - p)

        @pl.when(i >= 2)
        def _():
            scat_wait(p)

        for o in (0, 16, 32, 48, 64):
            dsc_v.at[p][pl.ds(o, 16)] = dst_v.at[p][pl.ds(o, 16)]
        compute(p)
        scat_issue(p)
        lin_issue(i + 2, p)

    def pair(k, carry):
        step(2 * k, 0)
        step(2 * k + 1, 1)
        return carry

    lax.fori_loop(0, _NCH // 2, pair, 0, unroll=False)
    step(_NCH - 1, 0)
    scat_wait(1)
    scat_wait(0)
    plsc.subcore_barrier()

    pltpu.sync_copy(agg_sh.at[pl.ds(base, 640)], aggp.at[core, pl.ds(base, 640)])
    pltpu.sync_copy(den_sh.at[pl.ds(base, 640)], denp.at[core, pl.ds(base, 640)])


def _edge_pass(S1, S2, se16, edge_feats, src, dst, cvec, zagg, zden):
    mesh = plsc.VectorSubcoreMesh(core_axis_name="c", subcore_axis_name="s")
    return pl.kernel(
        _edge_pass_body,
        out_type=[
            jax.ShapeDtypeStruct((2, _NP, 128), jnp.float32),
            jax.ShapeDtypeStruct((2, _NP, 16), jnp.float32),
        ],
        mesh=mesh,
        compiler_params=pltpu.CompilerParams(use_tc_tiling_on_sc=False),
        scratch_types=[
            pltpu.VMEM((2, _BC), jnp.int32),
            pltpu.VMEM((2, _BC), jnp.int32),
            pltpu.VMEM((2, _BC), jnp.int32),
            pltpu.VMEM((2, _BC, 16), jnp.float32),
            pltpu.VMEM((2, _BC, 16), jnp.float32),
            pltpu.VMEM((2, _BC, 16), jnp.float32),
            pltpu.VMEM((2, _BC, 16), jnp.float32),
            pltpu.VMEM((2, _BC, 16), jnp.float32),
            pltpu.VMEM((2, _BC, 128), jnp.float32),
            pltpu.VMEM((16,), jnp.float32),
            pltpu.VMEM_SHARED((_NP, 128), jnp.float32),
            pltpu.VMEM_SHARED((_NP, 16), jnp.float32),
            pltpu.SemaphoreType.DMA,
            pltpu.SemaphoreType.DMA,
            pltpu.SemaphoreType.DMA,
            pltpu.SemaphoreType.DMA,
            pltpu.SemaphoreType.DMA,
        ],
    )(S1, S2, se16, edge_feats, src, dst, cvec, zagg, zden)


def _combine_body(aggp_ref, denp_ref, zn_ref, r_ref, bd_ref, wot_ref, out_ref):
    agg = aggp_ref[0] + aggp_ref[1]
    den = denp_ref[0] + denp_ref[1]
    denr = jnp.dot(den, r_ref[...], preferred_element_type=jnp.float32)
    mask = denr > 0.0
    den_safe = jnp.where(mask, denr, 1.0)
    zz = jnp.dot(agg / den_safe, bd_ref[...], preferred_element_type=jnp.float32)
    z = jnp.where(mask, zz, zn_ref[...])
    out = jnp.dot(z, wot_ref[...], preferred_element_type=jnp.float32)
    out_ref[...] = jnp.maximum(out, 0.0)


def _combine(aggp, denp, z_n, R, BD, WoT):
    bn = 2000
    grid = (_N // bn,)
    return pl.pallas_call(
        _combine_body,
        grid=grid,
        in_specs=[
            pl.BlockSpec((2, bn, 128), lambda i: (0, i, 0)),
            pl.BlockSpec((2, bn, 16), lambda i: (0, i, 0)),
            pl.BlockSpec((bn, 128), lambda i: (i, 0)),
            pl.BlockSpec((16, 128), lambda i: (0, 0)),
            pl.BlockSpec((128, 128), lambda i: (0, 0)),
            pl.BlockSpec((128, 16), lambda i: (0, 0)),
        ],
        out_specs=pl.BlockSpec((bn, 16), lambda i: (i, 0)),
        out_shape=jax.ShapeDtypeStruct((_N, 16), jnp.float32),
    )(aggp, denp, z_n, R, BD, WoT)


def kernel(node_feats, edge_feats, edge_index, Wn, We, Wa, Wo):
    a1 = Wa[0, :_D]
    a2 = Wa[0, _D:]
    eye = jnp.eye(_H, dtype=jnp.float32)
    B1 = jnp.kron(eye, a1[:, None])
    B2 = jnp.kron(eye, a2[:, None])
    B12 = jnp.kron(eye, (a1 + a2)[:, None])
    pad = jnp.zeros((128, 8), jnp.float32)
    B1p = jnp.concatenate([B1, pad], axis=1)
    B2p = jnp.concatenate([B2, pad], axis=1)
    WeT = We.T
    Wse = WeT @ jnp.concatenate([B12, pad], axis=1)

    z_n, S1, S2 = _node_proj(node_feats, Wn.T, B1p, B2p)
    se16 = _edge_proj(edge_feats, Wse)

    c8 = jnp.maximum(jnp.max(S1[:, :_H], axis=0) + jnp.max(S2[:, :_H], axis=0)
                     + jnp.max(se16[:, :_H], axis=0), 0.0)
    cvec = jnp.concatenate([c8, jnp.full((8,), 1e30, jnp.float32)])

    zagg = jnp.zeros((640, 128), jnp.float32)
    zden = jnp.zeros((640, 16), jnp.float32)
    aggp, denp = _edge_pass(S1, S2, se16, edge_feats,
                            edge_index[0], edge_index[1], cvec, zagg, zden)

    R = jnp.kron(jnp.concatenate([eye, jnp.zeros((8, 8), jnp.float32)], axis=0),
                 jnp.ones((1, 16), jnp.float32))
    blockmask = jnp.kron(eye, jnp.ones((16, 16), jnp.float32))
    BD = jnp.tile(WeT, (_H, 1)) * blockmask
    return _combine(aggp[:, :_N], denp[:, :_N], z_n, R, BD, Wo.T)
